# baseline (speedup 1.0000x reference)
import jax
import jax.numpy as jnp
from jax import lax
from jax.experimental import pallas as pl
from jax.experimental.pallas import tpu as pltpu
from jax.experimental.pallas import tpu_sc as plsc

VOCAB = 100000
EMBED_DIM = 64
B = 4096
T = 200
N = B * T

NC = 2
NS = 16
NW = NC * NS

PER_B = B // NW
PER_W = PER_B * T
CHUNK_B = 2
STEPS = PER_B // CHUNK_B


def _fire_gather(table_hbm, idx_v, rows, sem, chunk_i):
    for k in range(CHUNK_B):
        pltpu.async_copy(
            table_hbm.at[idx_v.at[pl.ds((chunk_i * CHUNK_B + k) * T, T)]],
            rows.at[k],
            sem,
        )


def _wait_rows(table_hbm, rows, sem):
    for k in range(CHUNK_B):
        pltpu.make_async_copy(table_hbm.at[pl.ds(0, T)], rows.at[k], sem).wait()


def _fire_out(out_hbm, rows, sem, w_bbase, chunk_i):
    pltpu.async_copy(
        rows, out_hbm.at[pl.ds(w_bbase + chunk_i * CHUNK_B, CHUNK_B)], sem
    )


def _wait_out(out_hbm, rows, sem):
    pltpu.make_async_copy(rows, out_hbm.at[pl.ds(0, CHUNK_B)], sem).wait()


def _body(idx_hbm, table_hbm, out_hbm, idx_v, rows0, rows1, g0, g1, o0, o1):
    wid = lax.axis_index("s") * NC + lax.axis_index("c")
    w_base = wid * PER_W
    w_bbase = wid * PER_B
    rows = (rows0, rows1)
    gsem = (g0, g1)
    osem = (o0, o1)

    pltpu.sync_copy(idx_hbm.at[pl.ds(w_base, PER_W)], idx_v)

    _fire_gather(table_hbm, idx_v, rows[0], gsem[0], 0)
    _wait_rows(table_hbm, rows[0], gsem[0])
    _fire_out(out_hbm, rows[0], osem[0], w_bbase, 0)
    _fire_gather(table_hbm, idx_v, rows[1], gsem[1], 1)

    def slot(i, b):
        _wait_rows(table_hbm, rows[b], gsem[b])
        _fire_out(out_hbm, rows[b], osem[b], w_bbase, i)
        _wait_out(out_hbm, rows[1 - b], osem[1 - b])
        _fire_gather(table_hbm, idx_v, rows[1 - b], gsem[1 - b], i + 1)

    def pair(g, carry):
        slot(1 + 2 * g, 1)
        slot(2 + 2 * g, 0)
        return carry

    lax.fori_loop(0, (STEPS - 2) // 2, pair, 0)

    bl = (STEPS - 1) % 2
    _wait_rows(table_hbm, rows[bl], gsem[bl])
    _fire_out(out_hbm, rows[bl], osem[bl], w_bbase, STEPS - 1)
    _wait_out(out_hbm, rows[1 - bl], osem[1 - bl])
    _wait_out(out_hbm, rows[bl], osem[bl])


@jax.jit
def _embed(idx_flat, table):
    mesh = plsc.VectorSubcoreMesh(core_axis_name="c", subcore_axis_name="s")
    kern = pl.kernel(
        _body,
        out_type=jax.ShapeDtypeStruct((B, T, EMBED_DIM), jnp.float32),
        mesh=mesh,
        scratch_types=[
            pltpu.VMEM((PER_W,), jnp.int32),
            pltpu.VMEM((CHUNK_B, T, EMBED_DIM), jnp.float32),
            pltpu.VMEM((CHUNK_B, T, EMBED_DIM), jnp.float32),
            pltpu.SemaphoreType.DMA,
            pltpu.SemaphoreType.DMA,
            pltpu.SemaphoreType.DMA,
            pltpu.SemaphoreType.DMA,
        ],
        compiler_params=pltpu.CompilerParams(use_tc_tiling_on_sc=False),
    )
    return kern(idx_flat, table)


def kernel(action_tokens, table):
    idx_flat = action_tokens.reshape(-1).astype(jnp.int32)
    return _embed(idx_flat, table)
